# Initial kernel scaffold; baseline (speedup 1.0000x reference)
import jax
import jax.numpy as jnp
from jax.experimental import pallas as pl


def kernel(x, emb):
    raise NotImplementedError("write your pallas kernel here")


# speedup vs baseline: 1.4919x; 1.4919x over previous
import jax
import jax.numpy as jnp
from jax.experimental import pallas as pl


def _add_body(x_ref, e_ref, o_ref):
    o_ref[...] = x_ref[...] + e_ref[...][None]


def kernel(x, emb):
    B, S, D = x.shape
    TS = 512
    grid = (S // TS, B)
    out = pl.pallas_call(
        _add_body,
        grid=grid,
        in_specs=[
            pl.BlockSpec((1, TS, D), lambda i, j: (j, i, 0)),
            pl.BlockSpec((TS, D), lambda i, j: (i, 0)),
        ],
        out_specs=pl.BlockSpec((1, TS, D), lambda i, j: (j, i, 0)),
        out_shape=jax.ShapeDtypeStruct((B, S, D), x.dtype),
    )(x, emb)
    return out


# speedup vs baseline: 1.6661x; 1.1168x over previous
import jax
import jax.numpy as jnp
from jax.experimental import pallas as pl


def _add_body(x_ref, e_ref, o_ref):
    o_ref[...] = x_ref[...] + e_ref[...][None]


def kernel(x, emb):
    B, S, D = x.shape
    TS = 1024
    grid = (S // TS, B)
    out = pl.pallas_call(
        _add_body,
        grid=grid,
        in_specs=[
            pl.BlockSpec((1, TS, D), lambda i, j: (j, i, 0)),
            pl.BlockSpec((TS, D), lambda i, j: (i, 0)),
        ],
        out_specs=pl.BlockSpec((1, TS, D), lambda i, j: (j, i, 0)),
        out_shape=jax.ShapeDtypeStruct((B, S, D), x.dtype),
    )(x, emb)
    return out


# speedup vs baseline: 1.7385x; 1.0435x over previous
import jax
import jax.numpy as jnp
from jax.experimental import pallas as pl


def _add_body(x_ref, e_ref, o_ref):
    o_ref[...] = x_ref[...] + e_ref[...][None]


def kernel(x, emb):
    B, S, D = x.shape
    TS = 2048
    grid = (S // TS, B)
    out = pl.pallas_call(
        _add_body,
        grid=grid,
        in_specs=[
            pl.BlockSpec((1, TS, D), lambda i, j: (j, i, 0)),
            pl.BlockSpec((TS, D), lambda i, j: (i, 0)),
        ],
        out_specs=pl.BlockSpec((1, TS, D), lambda i, j: (j, i, 0)),
        out_shape=jax.ShapeDtypeStruct((B, S, D), x.dtype),
    )(x, emb)
    return out
